# baseline (speedup 1.0000x reference)
import functools

import jax
import jax.numpy as jnp
from jax import lax
from jax.experimental import pallas as pl
from jax.experimental.pallas import tpu as pltpu
from jax.experimental.pallas import tpu_sc as plsc

_E = 64
_TOPK = 2
_D = 768
_DFF = 1536
_N = 4096

_TM = 128
_NT = (_N * _TOPK) // _TM + _E
_NP = _NT * _TM
_DFB = 512
_NDF = _DFF // _DFB

_TB = 256


def _router_body(x_ref, gw_ref, gb_ref, i1_ref, i2_ref, s1_ref, s2_ref):
    logits = jnp.dot(x_ref[...], gw_ref[...],
                     preferred_element_type=jnp.float32) + gb_ref[...]
    cols = lax.broadcasted_iota(jnp.int32, logits.shape, 1)
    m1 = jnp.max(logits, axis=1, keepdims=True)
    i1 = jnp.min(jnp.where(logits == m1, cols, _E), axis=1, keepdims=True)
    masked = jnp.where(cols == i1, -jnp.inf, logits)
    m2 = jnp.max(masked, axis=1, keepdims=True)
    i2 = jnp.min(jnp.where(masked == m2, cols, _E), axis=1, keepdims=True)
    z = jnp.exp(m2 - m1)
    denom = 1.0 + z
    i1_ref[...] = i1
    i2_ref[...] = i2
    s1_ref[...] = 1.0 / denom
    s2_ref[...] = z / denom


def _route(x, gate_w, gate_b):
    n_blk = _N // _TB
    outs = pl.pallas_call(
        _router_body,
        grid=(n_blk,),
        in_specs=[
            pl.BlockSpec((_TB, _D), lambda t: (t, 0)),
            pl.BlockSpec((_D, _E), lambda t: (0, 0)),
            pl.BlockSpec((1, _E), lambda t: (0, 0)),
        ],
        out_specs=[
            pl.BlockSpec((_TB, 1), lambda t: (t, 0)),
            pl.BlockSpec((_TB, 1), lambda t: (t, 0)),
            pl.BlockSpec((_TB, 1), lambda t: (t, 0)),
            pl.BlockSpec((_TB, 1), lambda t: (t, 0)),
        ],
        out_shape=[
            jax.ShapeDtypeStruct((_N, 1), jnp.int32),
            jax.ShapeDtypeStruct((_N, 1), jnp.int32),
            jax.ShapeDtypeStruct((_N, 1), jnp.float32),
            jax.ShapeDtypeStruct((_N, 1), jnp.float32),
        ],
    )(x, gate_w, gate_b.reshape(1, _E))
    i1, i2, s1, s2 = outs
    top_i = jnp.concatenate([i1, i2], axis=1)
    score = jnp.concatenate([s1, s2], axis=1)
    return top_i, score


def _mlp_body(eot_ref, live_ref, x_ref, w1_ref, w2_ref, b1_ref, b2_ref,
              sc_ref, out_ref):
    t = pl.program_id(0)

    @pl.when(live_ref[t] != 0)
    def _compute():
        a = (jnp.dot(x_ref[...], w1_ref[0],
                     preferred_element_type=jnp.float32) + b1_ref[0])
        h = 0.5 * a * (1.0 + lax.erf(a * 0.7071067811865476))
        out_ref[...] = (jnp.dot(h, w2_ref[0],
                                preferred_element_type=jnp.float32)
                        + b2_ref[0]) * sc_ref[...]


def _expert_mlp(eot, live, x_padded, w1, w2, b1, b2, score_padded):
    grid_spec = pltpu.PrefetchScalarGridSpec(
        num_scalar_prefetch=2,
        grid=(_NT,),
        in_specs=[
            pl.BlockSpec((_TM, _D), lambda t, eot, live: (t, 0)),
            pl.BlockSpec((1, _D, _DFF), lambda t, eot, live: (eot[t], 0, 0)),
            pl.BlockSpec((1, _DFF, _D), lambda t, eot, live: (eot[t], 0, 0)),
            pl.BlockSpec((1, 1, _DFF), lambda t, eot, live: (eot[t], 0, 0)),
            pl.BlockSpec((1, 1, _D), lambda t, eot, live: (eot[t], 0, 0)),
            pl.BlockSpec((_TM, 1), lambda t, eot, live: (t, 0)),
        ],
        out_specs=pl.BlockSpec((_TM, _D), lambda t, eot, live: (t, 0)),
    )
    return pl.pallas_call(
        _mlp_body,
        grid_spec=grid_spec,
        out_shape=jax.ShapeDtypeStruct((_NP, _D), jnp.float32),
        compiler_params=pltpu.CompilerParams(
            dimension_semantics=("arbitrary",)),
    )(eot, live, x_padded, w1, w2, b1.reshape(_E, 1, _DFF),
      b2.reshape(_E, 1, _D), score_padded)


_NC = 2
_NS = 16
_NW = _NC * _NS

_GCHUNK = 64


def _sc_gather_rows(table, idx, n_rows):
    rows_per_w = n_rows // _NW
    n_chunks = rows_per_w // _GCHUNK
    d = table.shape[1]
    mesh = plsc.VectorSubcoreMesh(core_axis_name="c", subcore_axis_name="s")

    @functools.partial(
        pl.kernel, mesh=mesh,
        out_type=jax.ShapeDtypeStruct((n_rows, d), jnp.float32),
        scratch_types=[
            pltpu.VMEM((_GCHUNK,), jnp.int32),
            pltpu.VMEM((_GCHUNK, d), jnp.float32),
            pltpu.SemaphoreType.DMA,
        ],
    )
    def _gather(table_hbm, idx_hbm, out_hbm, idx_v, rows_v, sem):
        wid = lax.axis_index("s") * _NC + lax.axis_index("c")
        base = wid * rows_per_w

        def body(i, _):
            off = base + i * _GCHUNK
            pltpu.sync_copy(idx_hbm.at[pl.ds(off, _GCHUNK)], idx_v)
            pltpu.async_copy(table_hbm.at[idx_v], rows_v, sem).wait()
            pltpu.sync_copy(rows_v, out_hbm.at[pl.ds(off, _GCHUNK)])
            return _

        lax.fori_loop(0, n_chunks, body, 0)

    return _gather(table, idx)


_CCHUNK = 32


def _sc_combine(y, l0, l1):
    toks_per_w = _N // _NW
    n_chunks = toks_per_w // _CCHUNK
    mesh = plsc.VectorSubcoreMesh(core_axis_name="c", subcore_axis_name="s")

    @functools.partial(
        pl.kernel, mesh=mesh,
        out_type=jax.ShapeDtypeStruct((_N, _D), jnp.float32),
        scratch_types=[
            pltpu.VMEM((_CCHUNK,), jnp.int32),
            pltpu.VMEM((_CCHUNK,), jnp.int32),
            pltpu.VMEM((_CCHUNK, _D), jnp.float32),
            pltpu.VMEM((_CCHUNK, _D), jnp.float32),
            pltpu.SemaphoreType.DMA,
            pltpu.SemaphoreType.DMA,
        ],
    )
    def _combine(y_hbm, l0_hbm, l1_hbm, out_hbm, i0_v, i1_v, a_v, b_v,
                 sem0, sem1):
        wid = lax.axis_index("s") * _NC + lax.axis_index("c")
        base = wid * toks_per_w

        def body(i, _):
            off = base + i * _CCHUNK
            pltpu.sync_copy(l0_hbm.at[pl.ds(off, _CCHUNK)], i0_v)
            pltpu.sync_copy(l1_hbm.at[pl.ds(off, _CCHUNK)], i1_v)
            cp0 = pltpu.async_copy(y_hbm.at[i0_v], a_v, sem0)
            cp1 = pltpu.async_copy(y_hbm.at[i1_v], b_v, sem1)
            cp0.wait()
            cp1.wait()

            def row(r, _):
                for j in range(_D // 16):
                    sl = pl.ds(j * 16, 16)
                    a_v[r, sl] = a_v[r, sl] + b_v[r, sl]
                return _

            lax.fori_loop(0, _CCHUNK, row, 0)
            pltpu.sync_copy(a_v, out_hbm.at[pl.ds(off, _CCHUNK)])
            return _

        lax.fori_loop(0, n_chunks, body, 0)

    return _combine(y, l0, l1)


def kernel(x, gate_w, gate_b, w1, b1, w2, b2):
    top_i, score = _route(x, gate_w, gate_b)

    nc = _N * _TOPK
    flat_e = top_i.reshape(-1)
    flat_s = score.reshape(-1)
    order = jnp.argsort(flat_e)
    sorted_e = flat_e[order]
    counts = jnp.bincount(flat_e, length=_E)
    tiles = (counts + _TM - 1) // _TM
    tile_ends = jnp.cumsum(tiles)
    pstart = _TM * (tile_ends - tiles)
    starts = jnp.cumsum(counts) - counts
    pos_sorted = jnp.arange(nc, dtype=jnp.int32) - starts[sorted_e]
    ploc_sorted = (pstart[sorted_e] + pos_sorted).astype(jnp.int32)
    tok_sorted = (order // _TOPK).astype(jnp.int32)
    tok_padded = jnp.zeros((_NP,), jnp.int32).at[ploc_sorted].set(tok_sorted)
    ploc = jnp.zeros((nc,), jnp.int32).at[order].set(ploc_sorted)
    score_padded = jnp.zeros((_NP,), jnp.float32).at[ploc_sorted].set(
        flat_s[order])
    tidx = jnp.arange(_NT)
    eot = jnp.clip(
        jnp.searchsorted(tile_ends, tidx, side='right'),
        0, _E - 1).astype(jnp.int32)
    live = (tidx < tile_ends[-1]).astype(jnp.int32)

    x_padded = _sc_gather_rows(x, tok_padded, _NP)
    y = _expert_mlp(eot, live, x_padded, w1, w2, b1, b2,
                    score_padded.reshape(_NP, 1))
    ploc2 = ploc.reshape(_N, _TOPK)
    return y[ploc2[:, 0]] + y[ploc2[:, 1]]


# speedup vs baseline: 1.5826x; 1.5826x over previous
import functools

import jax
import jax.numpy as jnp
from jax import lax
from jax.experimental import pallas as pl
from jax.experimental.pallas import tpu as pltpu
from jax.experimental.pallas import tpu_sc as plsc

_E = 64
_TOPK = 2
_D = 768
_DFF = 1536
_N = 4096

_TM = 128
_NT = (_N * _TOPK) // _TM + _E
_NP = _NT * _TM
_DFB = 512
_NDF = _DFF // _DFB

_TB = 256


def _router_body(x_ref, gw_ref, gb_ref, i1_ref, i2_ref, s1_ref, s2_ref):
    logits = jnp.dot(x_ref[...], gw_ref[...],
                     preferred_element_type=jnp.float32) + gb_ref[...]
    cols = lax.broadcasted_iota(jnp.int32, logits.shape, 1)
    m1 = jnp.max(logits, axis=1, keepdims=True)
    i1 = jnp.min(jnp.where(logits == m1, cols, _E), axis=1, keepdims=True)
    masked = jnp.where(cols == i1, -jnp.inf, logits)
    m2 = jnp.max(masked, axis=1, keepdims=True)
    i2 = jnp.min(jnp.where(masked == m2, cols, _E), axis=1, keepdims=True)
    z = jnp.exp(m2 - m1)
    denom = 1.0 + z
    i1_ref[...] = i1
    i2_ref[...] = i2
    s1_ref[...] = 1.0 / denom
    s2_ref[...] = z / denom


def _route(x, gate_w, gate_b):
    n_blk = _N // _TB
    outs = pl.pallas_call(
        _router_body,
        grid=(n_blk,),
        in_specs=[
            pl.BlockSpec((_TB, _D), lambda t: (t, 0)),
            pl.BlockSpec((_D, _E), lambda t: (0, 0)),
            pl.BlockSpec((1, _E), lambda t: (0, 0)),
        ],
        out_specs=[
            pl.BlockSpec((_TB, 1), lambda t: (t, 0)),
            pl.BlockSpec((_TB, 1), lambda t: (t, 0)),
            pl.BlockSpec((_TB, 1), lambda t: (t, 0)),
            pl.BlockSpec((_TB, 1), lambda t: (t, 0)),
        ],
        out_shape=[
            jax.ShapeDtypeStruct((_N, 1), jnp.int32),
            jax.ShapeDtypeStruct((_N, 1), jnp.int32),
            jax.ShapeDtypeStruct((_N, 1), jnp.float32),
            jax.ShapeDtypeStruct((_N, 1), jnp.float32),
        ],
    )(x, gate_w, gate_b.reshape(1, _E))
    i1, i2, s1, s2 = outs
    top_i = jnp.concatenate([i1, i2], axis=1)
    score = jnp.concatenate([s1, s2], axis=1)
    return top_i, score


def _mlp_body(eot_ref, live_ref, x_ref, w1_ref, w2_ref, b1_ref, b2_ref,
              sc_ref, out_ref):
    t = pl.program_id(0)

    @pl.when(live_ref[t] != 0)
    def _compute():
        a = (jnp.dot(x_ref[...], w1_ref[0],
                     preferred_element_type=jnp.float32) + b1_ref[0])
        h = 0.5 * a * (1.0 + lax.erf(a * 0.7071067811865476))
        out_ref[...] = (jnp.dot(h, w2_ref[0],
                                preferred_element_type=jnp.float32)
                        + b2_ref[0]) * sc_ref[...]


def _expert_mlp(eot, live, x_padded, w1, w2, b1, b2, score_padded):
    grid_spec = pltpu.PrefetchScalarGridSpec(
        num_scalar_prefetch=2,
        grid=(_NT,),
        in_specs=[
            pl.BlockSpec((_TM, _D), lambda t, eot, live: (t, 0)),
            pl.BlockSpec((1, _D, _DFF), lambda t, eot, live: (eot[t], 0, 0)),
            pl.BlockSpec((1, _DFF, _D), lambda t, eot, live: (eot[t], 0, 0)),
            pl.BlockSpec((1, 1, _DFF), lambda t, eot, live: (eot[t], 0, 0)),
            pl.BlockSpec((1, 1, _D), lambda t, eot, live: (eot[t], 0, 0)),
            pl.BlockSpec((_TM, 1), lambda t, eot, live: (t, 0)),
        ],
        out_specs=pl.BlockSpec((_TM, _D), lambda t, eot, live: (t, 0)),
    )
    return pl.pallas_call(
        _mlp_body,
        grid_spec=grid_spec,
        out_shape=jax.ShapeDtypeStruct((_NP, _D), jnp.float32),
        compiler_params=pltpu.CompilerParams(
            dimension_semantics=("arbitrary",)),
    )(eot, live, x_padded, w1, w2, b1.reshape(_E, 1, _DFF),
      b2.reshape(_E, 1, _D), score_padded)


_NC = 2
_NS = 16
_NW = _NC * _NS

_GCHUNK = 64


def _sc_gather_rows(table, idx, n_rows):
    rows_per_w = n_rows // _NW
    n_chunks = rows_per_w // _GCHUNK
    d = table.shape[1]
    mesh = plsc.VectorSubcoreMesh(core_axis_name="c", subcore_axis_name="s")

    @functools.partial(
        pl.kernel, mesh=mesh,
        out_type=jax.ShapeDtypeStruct((n_rows, d), jnp.float32),
        scratch_types=[
            pltpu.VMEM((_GCHUNK,), jnp.int32),
            pltpu.VMEM((_GCHUNK, d), jnp.float32),
            pltpu.SemaphoreType.DMA,
        ],
    )
    def _gather(table_hbm, idx_hbm, out_hbm, idx_v, rows_v, sem):
        wid = lax.axis_index("s") * _NC + lax.axis_index("c")
        base = wid * rows_per_w

        def body(i, _):
            off = base + i * _GCHUNK
            pltpu.sync_copy(idx_hbm.at[pl.ds(off, _GCHUNK)], idx_v)
            pltpu.async_copy(table_hbm.at[idx_v], rows_v, sem).wait()
            pltpu.sync_copy(rows_v, out_hbm.at[pl.ds(off, _GCHUNK)])
            return _

        lax.fori_loop(0, n_chunks, body, 0)

    return _gather(table, idx)


_CCHUNK = 32


def _sc_combine(y, l0, l1):
    toks_per_w = _N // _NW
    n_chunks = toks_per_w // _CCHUNK
    mesh = plsc.VectorSubcoreMesh(core_axis_name="c", subcore_axis_name="s")

    @functools.partial(
        pl.kernel, mesh=mesh,
        out_type=jax.ShapeDtypeStruct((_N, _D), jnp.float32),
        scratch_types=[
            pltpu.VMEM((_CCHUNK,), jnp.int32),
            pltpu.VMEM((_CCHUNK,), jnp.int32),
            pltpu.VMEM((_CCHUNK, _D), jnp.float32),
            pltpu.VMEM((_CCHUNK, _D), jnp.float32),
            pltpu.SemaphoreType.DMA,
            pltpu.SemaphoreType.DMA,
        ],
    )
    def _combine(y_hbm, l0_hbm, l1_hbm, out_hbm, i0_v, i1_v, a_v, b_v,
                 sem0, sem1):
        wid = lax.axis_index("s") * _NC + lax.axis_index("c")
        base = wid * toks_per_w

        def body(i, _):
            off = base + i * _CCHUNK
            pltpu.sync_copy(l0_hbm.at[pl.ds(off, _CCHUNK)], i0_v)
            pltpu.sync_copy(l1_hbm.at[pl.ds(off, _CCHUNK)], i1_v)
            cp0 = pltpu.async_copy(y_hbm.at[i0_v], a_v, sem0)
            cp1 = pltpu.async_copy(y_hbm.at[i1_v], b_v, sem1)
            cp0.wait()
            cp1.wait()

            def row(r, _):
                for j in range(_D // 16):
                    sl = pl.ds(j * 16, 16)
                    a_v[r, sl] = a_v[r, sl] + b_v[r, sl]
                return _

            lax.fori_loop(0, _CCHUNK, row, 0)
            pltpu.sync_copy(a_v, out_hbm.at[pl.ds(off, _CCHUNK)])
            return _

        lax.fori_loop(0, n_chunks, body, 0)

    return _combine(y, l0, l1)


def kernel(x, gate_w, gate_b, w1, b1, w2, b2):
    top_i, score = _route(x, gate_w, gate_b)

    nc = _N * _TOPK
    flat_e = top_i.reshape(-1)
    flat_s = score.reshape(-1)
    order = jnp.argsort(flat_e)
    sorted_e = flat_e[order]
    counts = jnp.bincount(flat_e, length=_E)
    tiles = (counts + _TM - 1) // _TM
    tile_ends = jnp.cumsum(tiles)
    pstart = _TM * (tile_ends - tiles)
    starts = jnp.cumsum(counts) - counts
    pos_sorted = jnp.arange(nc, dtype=jnp.int32) - starts[sorted_e]
    ploc_sorted = (pstart[sorted_e] + pos_sorted).astype(jnp.int32)
    tok_sorted = (order // _TOPK).astype(jnp.int32)
    tok_padded = (jnp.arange(_NP, dtype=jnp.int32) % _N).at[
        ploc_sorted].set(tok_sorted)
    ploc = jnp.zeros((nc,), jnp.int32).at[order].set(ploc_sorted)
    score_padded = jnp.zeros((_NP,), jnp.float32).at[ploc_sorted].set(
        flat_s[order])
    tidx = jnp.arange(_NT)
    eot = jnp.clip(
        jnp.searchsorted(tile_ends, tidx, side='right'),
        0, _E - 1).astype(jnp.int32)
    live = (tidx < tile_ends[-1]).astype(jnp.int32)

    x_padded = _sc_gather_rows(x, tok_padded, _NP)
    y = _expert_mlp(eot, live, x_padded, w1, w2, b1, b2,
                    score_padded.reshape(_NP, 1))
    ploc2 = ploc.reshape(_N, _TOPK)
    return _sc_combine(y, ploc2[:, 0], ploc2[:, 1])
